# baseline (speedup 1.0000x reference)
import functools

import jax
import jax.numpy as jnp
from jax import lax
from jax.experimental import pallas as pl
from jax.experimental.pallas import tpu as pltpu
from jax.experimental.pallas import tpu_sc as plsc

N = 50000
IN_PLANES = 32
OUT_PLANES = 64
STRIDE = 4
NSAMPLE = 16
M = N // STRIDE

LANES = 128


def _fps_body(n_real, m, p3_ref, seed_ref, idx_ref, dists_ref, cur_ref):
    rows = p3_ref.shape[1]
    nch = rows // 8
    flat_iota = (jax.lax.broadcasted_iota(jnp.int32, (rows, LANES), 0) * LANES
                 + jax.lax.broadcasted_iota(jnp.int32, (rows, LANES), 1))
    dists_ref[...] = jnp.where(flat_iota < n_real, jnp.inf, -1.0).astype(jnp.float32)
    idx_ref[0] = jnp.int32(0)
    cur_ref[0] = seed_ref[0]
    cur_ref[1] = seed_ref[1]
    cur_ref[2] = seed_ref[2]
    iota8 = (jax.lax.broadcasted_iota(jnp.int32, (8, LANES), 0) * LANES
             + jax.lax.broadcasted_iota(jnp.int32, (8, LANES), 1))

    def merge(a, b):
        (va, ia), (vb, ib) = a, b
        take = (vb > va) | ((vb == va) & (ib < ia))
        return jnp.where(take, vb, va), jnp.where(take, ib, ia)

    def body(i, _):
        cx = cur_ref[0]
        cy = cur_ref[1]
        cz = cur_ref[2]
        champs = []
        for k in range(nch):
            sl = pl.ds(k * 8, 8)
            dx = p3_ref[0, sl, :] - cx
            dy = p3_ref[1, sl, :] - cy
            dz = p3_ref[2, sl, :] - cz
            d = (dx * dx + dz * dz) + dy * dy
            nd = jnp.minimum(dists_ref[sl, :], d)
            dists_ref[sl, :] = nd
            champs.append((nd, iota8 + k * (8 * LANES)))
        while len(champs) > 1:
            nxt = [merge(champs[j], champs[j + 1])
                   for j in range(0, len(champs) - 1, 2)]
            if len(champs) % 2:
                nxt.append(champs[-1])
            champs = nxt
        bv, bi = champs[0]
        bv, bi = merge((bv[:4], bi[:4]), (bv[4:], bi[4:]))
        bv, bi = merge((bv[:2], bi[:2]), (bv[2:], bi[2:]))
        bv, bi = merge((bv[:1], bi[:1]), (bv[1:], bi[1:]))
        for s in (64, 32, 16, 8, 4, 2, 1):
            bv, bi = merge((bv, bi), (pltpu.roll(bv, s, 1), pltpu.roll(bi, s, 1)))
        n_idx = jnp.min(bi)
        idx_ref[i] = n_idx
        row = n_idx // LANES
        lane = n_idx - row * LANES
        lane_mask = jax.lax.broadcasted_iota(jnp.int32, (1, LANES), 1) == lane
        neg_inf = jnp.float32(-jnp.inf)
        cur_ref[0] = jnp.max(jnp.where(lane_mask, p3_ref[0, pl.ds(row, 1), :], neg_inf))
        cur_ref[1] = jnp.max(jnp.where(lane_mask, p3_ref[1, pl.ds(row, 1), :], neg_inf))
        cur_ref[2] = jnp.max(jnp.where(lane_mask, p3_ref[2, pl.ds(row, 1), :], neg_inf))
        return 0

    jax.lax.fori_loop(1, m, body, 0)


def _fps_pallas(p, n_real, m, interpret=False):
    n_pad = ((n_real + 1023) // 1024) * 1024
    rows = n_pad // LANES
    m_pad = ((m + 1023) // 1024) * 1024
    pp = jnp.zeros((n_pad, 3), p.dtype).at[:n_real].set(p)
    p3 = pp.T.reshape(3, rows, LANES)
    seed = p[0]
    idx = pl.pallas_call(
        functools.partial(_fps_body, n_real, m),
        in_specs=[
            pl.BlockSpec(memory_space=pltpu.VMEM),
            pl.BlockSpec(memory_space=pltpu.SMEM),
        ],
        out_specs=pl.BlockSpec(memory_space=pltpu.SMEM),
        out_shape=jax.ShapeDtypeStruct((m_pad,), jnp.int32),
        scratch_shapes=[
            pltpu.VMEM((rows, LANES), jnp.float32),
            pltpu.SMEM((4,), jnp.float32),
        ],
        interpret=interpret,
    )(p3, seed)
    return idx[:m]


def _knn_body(nchunk, pos, cb, pxb_ref, pyb_ref, pzb_ref, p2_ref,
              cent_ref, res_ref, m1_ref, m2_ref, a1_ref, a2_ref):
    inf = jnp.float32(jnp.inf)
    big = jnp.int32(2**30)
    m1_ref[...] = jnp.full((cb, pos), inf, jnp.float32)
    m2_ref[...] = jnp.full((cb, pos), inf, jnp.float32)
    a1_ref[...] = jnp.full((cb, pos), big, jnp.int32)
    a2_ref[...] = jnp.full((cb, pos), big, jnp.int32)
    def bfr(v):
        return v.astype(jnp.bfloat16).astype(jnp.float32)

    cx = bfr(cent_ref[:, 0:1])
    cy = bfr(cent_ref[:, 1:2])
    cz = bfr(cent_ref[:, 2:3])
    c2 = cent_ref[:, 3:4]
    lane_iota = jax.lax.broadcasted_iota(jnp.int32, (1, pos), 1)

    def chunk(ch, _):
        px = bfr(pxb_ref[ch])
        py = bfr(pyb_ref[ch])
        pz = bfr(pzb_ref[ch])
        p2 = p2_ref[ch]
        t = (cx * px + cy * py) + cz * pz
        dd = (c2 + p2) - 2.0 * t
        ivec = jnp.broadcast_to(lane_iota + ch * pos, (cb, pos))
        m1 = m1_ref[...]
        m2 = m2_ref[...]
        a1 = a1_ref[...]
        a2 = a2_ref[...]
        c1 = dd < m1
        c2m = dd < m2
        a2_ref[...] = jnp.where(c1, a1, jnp.where(c2m, ivec, a2))
        m2_ref[...] = jnp.where(c1, m1, jnp.where(c2m, dd, m2))
        a1_ref[...] = jnp.where(c1, ivec, a1)
        m1_ref[...] = jnp.where(c1, dd, m1)
        return 0

    jax.lax.fori_loop(0, nchunk, chunk, 0)

    kcol = jax.lax.broadcasted_iota(jnp.int32, (cb, NSAMPLE), 1)

    def rnd(r, acc):
        m1 = m1_ref[...]
        m2 = m2_ref[...]
        a1 = a1_ref[...]
        a2 = a2_ref[...]
        vmin = jnp.minimum(jnp.min(m1, axis=1, keepdims=True),
                           jnp.min(m2, axis=1, keepdims=True))
        eq1 = m1 == vmin
        eq2 = m2 == vmin
        sel = jnp.minimum(
            jnp.min(jnp.where(eq1, a1, big), axis=1, keepdims=True),
            jnp.min(jnp.where(eq2, a2, big), axis=1, keepdims=True))
        acc = jnp.where(kcol == r, sel, acc)
        m1_ref[...] = jnp.where(eq1 & (a1 == sel), inf, m1)
        m2_ref[...] = jnp.where(eq2 & (a2 == sel), inf, m2)
        return acc

    res_ref[0] = jax.lax.fori_loop(
        0, NSAMPLE, rnd, jnp.zeros((cb, NSAMPLE), jnp.int32))


def _knn_pallas(n_p, p, pos=1024, cb=128, interpret=False):
    m, n = n_p.shape[0], p.shape[0]
    n_pad = ((n + pos - 1) // pos) * pos
    nchunk = n_pad // pos
    m_pad = ((m + cb - 1) // cb) * cb
    nstep = m_pad // cb

    pp = jnp.full((n_pad, 3), 1e6, p.dtype).at[:n].set(p)
    p2 = jnp.sum(pp * pp, axis=1)
    pxb = pp[:, 0].reshape(nchunk, 1, pos)
    pyb = pp[:, 1].reshape(nchunk, 1, pos)
    pzb = pp[:, 2].reshape(nchunk, 1, pos)
    p2r = p2.reshape(nchunk, 1, pos)

    cpad = jnp.zeros((m_pad, 3), n_p.dtype).at[:m].set(n_p)
    c2 = jnp.sum(cpad * cpad, axis=1)
    cent = jnp.concatenate(
        [cpad, c2[:, None], jnp.zeros((m_pad, 4), jnp.float32)], axis=1)

    const_spec = pl.BlockSpec((nchunk, 1, pos), lambda i: (0, 0, 0))
    res = pl.pallas_call(
        functools.partial(_knn_body, nchunk, pos, cb),
        grid=(nstep,),
        in_specs=[const_spec] * 4
        + [pl.BlockSpec((cb, 8), lambda i: (i, 0))],
        out_specs=pl.BlockSpec((1, cb, NSAMPLE), lambda i: (i, 0, 0)),
        out_shape=jax.ShapeDtypeStruct((nstep, cb, NSAMPLE), jnp.int32),
        scratch_shapes=[pltpu.VMEM((cb, pos), jnp.float32)] * 2
        + [pltpu.VMEM((cb, pos), jnp.int32)] * 2,
        interpret=interpret,
    )(pxb, pyb, pzb, p2r, cent)
    return res.reshape(m_pad, NSAMPLE)


def _ymat_body(x_ref, w_ref, p_ref, y_ref):
    xb = x_ref[...].astype(jnp.bfloat16)
    wb = w_ref[...].astype(jnp.bfloat16)
    blk = x_ref.shape[0]
    y_ref[:, 0:OUT_PLANES] = jnp.dot(xb, wb, preferred_element_type=jnp.float32)
    y_ref[:, OUT_PLANES:OUT_PLANES + 16] = p_ref[...]
    y_ref[:, OUT_PLANES + 16:] = jnp.zeros((blk, 128 - OUT_PLANES - 16),
                                           jnp.float32)


def _ymat_pallas(xpad, Wx, prow):
    n_pad = xpad.shape[0]
    blk = 512
    return pl.pallas_call(
        _ymat_body,
        grid=(n_pad // blk,),
        in_specs=[pl.BlockSpec((blk, IN_PLANES), lambda i: (i, 0)),
                  pl.BlockSpec((IN_PLANES, OUT_PLANES), lambda i: (0, 0)),
                  pl.BlockSpec((blk, 16), lambda i: (i, 0))],
        out_specs=pl.BlockSpec((blk, 128), lambda i: (i, 0)),
        out_shape=jax.ShapeDtypeStruct((n_pad, 128), jnp.float32),
    )(xpad, Wx, prow)


def _splat(vec, q):
    dnums = lax.GatherDimensionNumbers(
        offset_dims=(), collapsed_slice_dims=(0,), start_index_map=(0,))
    idx = jnp.full((16, 1), q, jnp.int32)
    return lax.gather(vec, idx, dnums, (1,),
                      mode=lax.GatherScatterMode.PROMISE_IN_BOUNDS)


_SC_BC = 8
_SC_NB = 49


def _scpool_body(idx_hbm, y_hbm, cent_hbm, wxyz_hbm, agg_hbm,
                 idx_v, y_v, c_v, w_v, agg_v, sem1):
    nc = 2
    wid = lax.axis_index("s") * nc + lax.axis_index("c")

    def bfr(v):
        return v.astype(jnp.bfloat16).astype(jnp.float32)

    pltpu.sync_copy(wxyz_hbm, w_v)
    wvec = [[bfr(w_v[q, pl.ds(v * 16, 16)]) for v in range(4)] for q in range(3)]
    rows_per_batch = _SC_BC * NSAMPLE
    neg_inf = jnp.float32(-jnp.inf)

    def batch(b, _):
        base_c = wid * (_SC_NB * _SC_BC) + b * _SC_BC
        pltpu.sync_copy(idx_hbm.at[pl.ds(base_c * NSAMPLE, rows_per_batch)],
                        idx_v)
        cp1 = pltpu.async_copy(y_hbm.at[idx_v], y_v, sem1)
        pltpu.sync_copy(cent_hbm.at[pl.ds(base_c, _SC_BC)], c_v)
        cp1.wait()

        def center(ci, _):
            cvec = c_v[ci, pl.ds(0, 16)]
            amax = [jnp.full((16,), neg_inf, jnp.float32) for _ in range(4)]
            amin = [jnp.full((16,), -neg_inf, jnp.float32) for _ in range(4)]
            asum = [jnp.zeros((16,), jnp.float32) for _ in range(4)]
            asq = [jnp.zeros((16,), jnp.float32) for _ in range(4)]
            for n in range(NSAMPLE):
                r = ci * NSAMPLE + n
                db = bfr(y_v[r, pl.ds(OUT_PLANES, 16)] - cvec)
                s0 = _splat(db, 0)
                s1 = _splat(db, 1)
                s2 = _splat(db, 2)
                for v in range(4):
                    h = (y_v[r, pl.ds(v * 16, 16)]
                         + (s0 * wvec[0][v] + s1 * wvec[1][v])
                         + s2 * wvec[2][v])
                    amax[v] = jnp.maximum(amax[v], h)
                    amin[v] = jnp.minimum(amin[v], h)
                    asum[v] = asum[v] + h
                    asq[v] = asq[v] + h * h
            for v in range(4):
                agg_v[ci, pl.ds(v * 16, 16)] = amax[v]
                agg_v[ci, pl.ds(64 + v * 16, 16)] = amin[v]
                agg_v[ci, pl.ds(128 + v * 16, 16)] = asum[v]
                agg_v[ci, pl.ds(192 + v * 16, 16)] = asq[v]
            return 0

        lax.fori_loop(0, _SC_BC, center, 0)
        pltpu.sync_copy(agg_v, agg_hbm.at[pl.ds(base_c, _SC_BC)])
        return 0

    lax.fori_loop(0, _SC_NB, batch, 0)


def _scpool_pallas(knn_flat, ytab, cent, wxyz):
    m_pad = cent.shape[0]
    mesh = plsc.VectorSubcoreMesh(core_axis_name="c", subcore_axis_name="s")
    f = pl.kernel(
        _scpool_body,
        mesh=mesh,
        out_type=jax.ShapeDtypeStruct((m_pad, 256), jnp.float32),
        scratch_types=[
            pltpu.VMEM((_SC_BC * NSAMPLE,), jnp.int32),
            pltpu.VMEM((_SC_BC * NSAMPLE, 128), jnp.float32),
            pltpu.VMEM((_SC_BC, 128), jnp.float32),
            pltpu.VMEM((8, 128), jnp.float32),
            pltpu.VMEM((_SC_BC, 256), jnp.float32),
            pltpu.SemaphoreType.DMA,
        ],
    )
    return f(knn_flat, ytab, cent, wxyz)


def _final_body(n_valid, agg_ref, g_ref, b_ref, out_ref):
    m_pad = agg_ref.shape[0]
    rows = jax.lax.broadcasted_iota(jnp.int32, (m_pad, OUT_PLANES), 0) < n_valid
    zero = jnp.float32(0.0)
    hsum = jnp.where(rows, agg_ref[:, 128:192], zero)
    hsq = jnp.where(rows, agg_ref[:, 192:256], zero)
    cnt = jnp.float32(n_valid * NSAMPLE)
    mean = jnp.sum(hsum, axis=0, keepdims=True) / cnt
    var = jnp.sum(hsq, axis=0, keepdims=True) / cnt - mean * mean
    sq = jnp.sqrt(var + 1e-5)
    g = g_ref[...]
    v = jnp.where(g >= 0, agg_ref[:, 0:64], agg_ref[:, 64:128])
    out_ref[...] = jnp.maximum((v - mean) / sq * g + b_ref[...], zero)


def _final_pallas(agg, gamma, beta, n_valid):
    m_pad = agg.shape[0]
    return pl.pallas_call(
        functools.partial(_final_body, n_valid),
        in_specs=[pl.BlockSpec((m_pad, 256), lambda: (0, 0)),
                  pl.BlockSpec((1, OUT_PLANES), lambda: (0, 0)),
                  pl.BlockSpec((1, OUT_PLANES), lambda: (0, 0))],
        out_specs=pl.BlockSpec((m_pad, OUT_PLANES), lambda: (0, 0)),
        out_shape=jax.ShapeDtypeStruct((m_pad, OUT_PLANES), jnp.float32),
    )(agg, gamma.reshape(1, -1), beta.reshape(1, -1))


def kernel(p, x, o, W, gamma, beta):
    idx = _fps_pallas(p, N, M)
    n_p = p[idx]
    knn_full = _knn_pallas(n_p, p)

    n_pad = 50176
    m_pad = knn_full.shape[0]
    xpad = jnp.zeros((n_pad, IN_PLANES), jnp.float32).at[:N].set(x)
    prow = jnp.zeros((n_pad, 16), jnp.float32).at[:N, 0:3].set(p)
    ytab = _ymat_pallas(xpad, W[3:], prow)
    cent = jnp.zeros((m_pad, 128), jnp.float32).at[:M, 0:3].set(n_p)
    wxyz = jnp.zeros((8, 128), jnp.float32).at[0:3, 0:OUT_PLANES].set(W[0:3])
    agg = _scpool_pallas(knn_full.reshape(-1), ytab, cent, wxyz)
    out = _final_pallas(agg, gamma, beta, M)[:M]
    n_o = jnp.array([M], dtype=jnp.int32)
    return (n_p, out, n_o)


# speedup vs baseline: 1.4678x; 1.4678x over previous
import functools

import jax
import jax.numpy as jnp
from jax import lax
from jax.experimental import pallas as pl
from jax.experimental.pallas import tpu as pltpu
from jax.experimental.pallas import tpu_sc as plsc

N = 50000
IN_PLANES = 32
OUT_PLANES = 64
STRIDE = 4
NSAMPLE = 16
M = N // STRIDE

LANES = 128


def _fps_body(n_real, m, p3_ref, seed_ref, idx_ref, dists_ref, iota_ref,
              cur_ref):
    rows = p3_ref.shape[1]
    nch = rows // 8
    flat_iota = (jax.lax.broadcasted_iota(jnp.int32, (rows, LANES), 0) * LANES
                 + jax.lax.broadcasted_iota(jnp.int32, (rows, LANES), 1))
    iota_ref[...] = flat_iota
    dists_ref[...] = jnp.where(flat_iota < n_real, jnp.inf, -1.0).astype(jnp.float32)
    sx = seed_ref[0]
    sy = seed_ref[1]
    sz = seed_ref[2]
    cur_ref[0:1, :] = jnp.full((1, LANES), sx, jnp.float32)
    cur_ref[1:2, :] = jnp.full((1, LANES), sy, jnp.float32)
    cur_ref[2:3, :] = jnp.full((1, LANES), sz, jnp.float32)
    lane_iota = jax.lax.broadcasted_iota(jnp.int32, (1, LANES), 1)
    neg_inf = jnp.float32(-jnp.inf)

    def tree(op, arrs):
        while len(arrs) > 1:
            nxt = [op(arrs[j], arrs[j + 1])
                   for j in range(0, len(arrs) - 1, 2)]
            if len(arrs) % 2:
                nxt.append(arrs[-1])
            arrs = nxt
        return arrs[0]

    def allreduce(op, a8):
        a = op(a8[:4], a8[4:])
        a = op(a[:2], a[2:])
        a = op(a[:1], a[1:])
        for s in (64, 32, 16, 8, 4, 2, 1):
            a = op(a, pltpu.roll(a, s, 1))
        return a

    def body(i, buf):
        cx = cur_ref[0:1, :]
        cy = cur_ref[1:2, :]
        cz = cur_ref[2:3, :]
        dx = p3_ref[0] - cx
        dy = p3_ref[1] - cy
        dz = p3_ref[2] - cz
        d = (dx * dx + dz * dz) + dy * dy
        dists = jnp.minimum(dists_ref[...], d)
        dists_ref[...] = dists
        vmax8 = tree(jnp.maximum,
                     [dists[k * 8:(k + 1) * 8] for k in range(nch)])
        vmax = jnp.max(vmax8)
        cand = jnp.where(dists == vmax, iota_ref[...], jnp.int32(2**30))
        imin8 = tree(jnp.minimum,
                     [cand[k * 8:(k + 1) * 8] for k in range(nch)])
        n_idx = jnp.min(imin8)
        row = n_idx // LANES
        lane = n_idx - row * LANES
        lane_mask = lane_iota == lane
        cur_ref[0:1, :] = jnp.full((1, LANES), jnp.max(
            jnp.where(lane_mask, p3_ref[0, pl.ds(row, 1), :], neg_inf)),
            jnp.float32)
        cur_ref[1:2, :] = jnp.full((1, LANES), jnp.max(
            jnp.where(lane_mask, p3_ref[1, pl.ds(row, 1), :], neg_inf)),
            jnp.float32)
        cur_ref[2:3, :] = jnp.full((1, LANES), jnp.max(
            jnp.where(lane_mask, p3_ref[2, pl.ds(row, 1), :], neg_inf)),
            jnp.float32)
        buf = jnp.where(lane_iota == (i & (LANES - 1)), n_idx, buf)

        @pl.when(((i & (LANES - 1)) == (LANES - 1)) | (i == m - 1))
        def _():
            idx_ref[pl.ds(i // LANES, 1), :] = buf

        return buf

    buf0 = jnp.zeros((1, LANES), jnp.int32)
    jax.lax.fori_loop(1, m, body, buf0)


def _fps_pallas(p, n_real, m, interpret=False):
    n_pad = ((n_real + 1023) // 1024) * 1024
    rows = n_pad // LANES
    m_pad = ((m + 1023) // 1024) * 1024
    pp = jnp.zeros((n_pad, 3), p.dtype).at[:n_real].set(p)
    p3 = pp.T.reshape(3, rows, LANES)
    seed = p[0]
    idx = pl.pallas_call(
        functools.partial(_fps_body, n_real, m),
        in_specs=[
            pl.BlockSpec(memory_space=pltpu.VMEM),
            pl.BlockSpec(memory_space=pltpu.SMEM),
        ],
        out_specs=pl.BlockSpec(memory_space=pltpu.VMEM),
        out_shape=jax.ShapeDtypeStruct((m_pad // LANES, LANES), jnp.int32),
        scratch_shapes=[
            pltpu.VMEM((rows, LANES), jnp.float32),
            pltpu.VMEM((rows, LANES), jnp.int32),
            pltpu.VMEM((8, LANES), jnp.float32),
        ],
        interpret=interpret,
    )(p3, seed)
    return idx.reshape(m_pad)[:m]


def _knn_body(nchunk, pos, cb, pxb_ref, pyb_ref, pzb_ref, p2_ref,
              cent_ref, res_ref, m1_ref, m2_ref, a1_ref, a2_ref):
    inf = jnp.float32(jnp.inf)
    big = jnp.int32(2**30)
    m1_ref[...] = jnp.full((cb, pos), inf, jnp.float32)
    m2_ref[...] = jnp.full((cb, pos), inf, jnp.float32)
    a1_ref[...] = jnp.full((cb, pos), big, jnp.int32)
    a2_ref[...] = jnp.full((cb, pos), big, jnp.int32)
    def bfr(v):
        return v.astype(jnp.bfloat16).astype(jnp.float32)

    cx = bfr(cent_ref[:, 0:1])
    cy = bfr(cent_ref[:, 1:2])
    cz = bfr(cent_ref[:, 2:3])
    c2 = cent_ref[:, 3:4]
    lane_iota = jax.lax.broadcasted_iota(jnp.int32, (1, pos), 1)

    def chunk(ch, _):
        px = bfr(pxb_ref[ch])
        py = bfr(pyb_ref[ch])
        pz = bfr(pzb_ref[ch])
        p2 = p2_ref[ch]
        t = (cx * px + cy * py) + cz * pz
        dd = (c2 + p2) - 2.0 * t
        ivec = jnp.broadcast_to(lane_iota + ch * pos, (cb, pos))
        m1 = m1_ref[...]
        m2 = m2_ref[...]
        a1 = a1_ref[...]
        a2 = a2_ref[...]
        c1 = dd < m1
        c2m = dd < m2
        a2_ref[...] = jnp.where(c1, a1, jnp.where(c2m, ivec, a2))
        m2_ref[...] = jnp.where(c1, m1, jnp.where(c2m, dd, m2))
        a1_ref[...] = jnp.where(c1, ivec, a1)
        m1_ref[...] = jnp.where(c1, dd, m1)
        return 0

    jax.lax.fori_loop(0, nchunk, chunk, 0)

    kcol = jax.lax.broadcasted_iota(jnp.int32, (cb, NSAMPLE), 1)

    def rnd(r, acc):
        m1 = m1_ref[...]
        m2 = m2_ref[...]
        a1 = a1_ref[...]
        a2 = a2_ref[...]
        vmin = jnp.minimum(jnp.min(m1, axis=1, keepdims=True),
                           jnp.min(m2, axis=1, keepdims=True))
        eq1 = m1 == vmin
        eq2 = m2 == vmin
        sel = jnp.minimum(
            jnp.min(jnp.where(eq1, a1, big), axis=1, keepdims=True),
            jnp.min(jnp.where(eq2, a2, big), axis=1, keepdims=True))
        acc = jnp.where(kcol == r, sel, acc)
        m1_ref[...] = jnp.where(eq1 & (a1 == sel), inf, m1)
        m2_ref[...] = jnp.where(eq2 & (a2 == sel), inf, m2)
        return acc

    res_ref[0] = jax.lax.fori_loop(
        0, NSAMPLE, rnd, jnp.zeros((cb, NSAMPLE), jnp.int32))


def _knn_pallas(n_p, p, pos=1024, cb=128, interpret=False):
    m, n = n_p.shape[0], p.shape[0]
    n_pad = ((n + pos - 1) // pos) * pos
    nchunk = n_pad // pos
    m_pad = ((m + cb - 1) // cb) * cb
    nstep = m_pad // cb

    pp = jnp.full((n_pad, 3), 1e6, p.dtype).at[:n].set(p)
    p2 = jnp.sum(pp * pp, axis=1)
    pxb = pp[:, 0].reshape(nchunk, 1, pos)
    pyb = pp[:, 1].reshape(nchunk, 1, pos)
    pzb = pp[:, 2].reshape(nchunk, 1, pos)
    p2r = p2.reshape(nchunk, 1, pos)

    cpad = jnp.zeros((m_pad, 3), n_p.dtype).at[:m].set(n_p)
    c2 = jnp.sum(cpad * cpad, axis=1)
    cent = jnp.concatenate(
        [cpad, c2[:, None], jnp.zeros((m_pad, 4), jnp.float32)], axis=1)

    const_spec = pl.BlockSpec((nchunk, 1, pos), lambda i: (0, 0, 0))
    res = pl.pallas_call(
        functools.partial(_knn_body, nchunk, pos, cb),
        grid=(nstep,),
        in_specs=[const_spec] * 4
        + [pl.BlockSpec((cb, 8), lambda i: (i, 0))],
        out_specs=pl.BlockSpec((1, cb, NSAMPLE), lambda i: (i, 0, 0)),
        out_shape=jax.ShapeDtypeStruct((nstep, cb, NSAMPLE), jnp.int32),
        scratch_shapes=[pltpu.VMEM((cb, pos), jnp.float32)] * 2
        + [pltpu.VMEM((cb, pos), jnp.int32)] * 2,
        interpret=interpret,
    )(pxb, pyb, pzb, p2r, cent)
    return res.reshape(m_pad, NSAMPLE)


def _ymat_body(x_ref, w_ref, p_ref, y_ref):
    xb = x_ref[...].astype(jnp.bfloat16)
    wb = w_ref[...].astype(jnp.bfloat16)
    blk = x_ref.shape[0]
    y_ref[:, 0:OUT_PLANES] = jnp.dot(xb, wb, preferred_element_type=jnp.float32)
    y_ref[:, OUT_PLANES:OUT_PLANES + 16] = p_ref[...]
    y_ref[:, OUT_PLANES + 16:] = jnp.zeros((blk, 128 - OUT_PLANES - 16),
                                           jnp.float32)


def _ymat_pallas(xpad, Wx, prow):
    n_pad = xpad.shape[0]
    blk = 512
    return pl.pallas_call(
        _ymat_body,
        grid=(n_pad // blk,),
        in_specs=[pl.BlockSpec((blk, IN_PLANES), lambda i: (i, 0)),
                  pl.BlockSpec((IN_PLANES, OUT_PLANES), lambda i: (0, 0)),
                  pl.BlockSpec((blk, 16), lambda i: (i, 0))],
        out_specs=pl.BlockSpec((blk, 128), lambda i: (i, 0)),
        out_shape=jax.ShapeDtypeStruct((n_pad, 128), jnp.float32),
    )(xpad, Wx, prow)


def _splat(vec, q):
    dnums = lax.GatherDimensionNumbers(
        offset_dims=(), collapsed_slice_dims=(0,), start_index_map=(0,))
    idx = jnp.full((16, 1), q, jnp.int32)
    return lax.gather(vec, idx, dnums, (1,),
                      mode=lax.GatherScatterMode.PROMISE_IN_BOUNDS)


_SC_BC = 8
_SC_NB = 49


def _scpool_body(idx_hbm, y_hbm, cent_hbm, wxyz_hbm, agg_hbm,
                 idx_v, y_v, c_v, w_v, agg_v, sem1):
    nc = 2
    wid = lax.axis_index("s") * nc + lax.axis_index("c")

    def bfr(v):
        return v.astype(jnp.bfloat16).astype(jnp.float32)

    pltpu.sync_copy(wxyz_hbm, w_v)
    wvec = [[bfr(w_v[q, pl.ds(v * 16, 16)]) for v in range(4)] for q in range(3)]
    rows_per_batch = _SC_BC * NSAMPLE
    neg_inf = jnp.float32(-jnp.inf)

    def batch(b, _):
        base_c = wid * (_SC_NB * _SC_BC) + b * _SC_BC
        pltpu.sync_copy(idx_hbm.at[pl.ds(base_c * NSAMPLE, rows_per_batch)],
                        idx_v)
        cp1 = pltpu.async_copy(y_hbm.at[idx_v], y_v, sem1)
        pltpu.sync_copy(cent_hbm.at[pl.ds(base_c, _SC_BC)], c_v)
        cp1.wait()

        def center(ci, _):
            cvec = c_v[ci, pl.ds(0, 16)]
            amax = [jnp.full((16,), neg_inf, jnp.float32) for _ in range(4)]
            amin = [jnp.full((16,), -neg_inf, jnp.float32) for _ in range(4)]
            asum = [jnp.zeros((16,), jnp.float32) for _ in range(4)]
            asq = [jnp.zeros((16,), jnp.float32) for _ in range(4)]
            for n in range(NSAMPLE):
                r = ci * NSAMPLE + n
                db = bfr(y_v[r, pl.ds(OUT_PLANES, 16)] - cvec)
                s0 = _splat(db, 0)
                s1 = _splat(db, 1)
                s2 = _splat(db, 2)
                for v in range(4):
                    h = (y_v[r, pl.ds(v * 16, 16)]
                         + (s0 * wvec[0][v] + s1 * wvec[1][v])
                         + s2 * wvec[2][v])
                    amax[v] = jnp.maximum(amax[v], h)
                    amin[v] = jnp.minimum(amin[v], h)
                    asum[v] = asum[v] + h
                    asq[v] = asq[v] + h * h
            for v in range(4):
                agg_v[ci, pl.ds(v * 16, 16)] = amax[v]
                agg_v[ci, pl.ds(64 + v * 16, 16)] = amin[v]
                agg_v[ci, pl.ds(128 + v * 16, 16)] = asum[v]
                agg_v[ci, pl.ds(192 + v * 16, 16)] = asq[v]
            return 0

        lax.fori_loop(0, _SC_BC, center, 0)
        pltpu.sync_copy(agg_v, agg_hbm.at[pl.ds(base_c, _SC_BC)])
        return 0

    lax.fori_loop(0, _SC_NB, batch, 0)


def _scpool_pallas(knn_flat, ytab, cent, wxyz):
    m_pad = cent.shape[0]
    mesh = plsc.VectorSubcoreMesh(core_axis_name="c", subcore_axis_name="s")
    f = pl.kernel(
        _scpool_body,
        mesh=mesh,
        out_type=jax.ShapeDtypeStruct((m_pad, 256), jnp.float32),
        scratch_types=[
            pltpu.VMEM((_SC_BC * NSAMPLE,), jnp.int32),
            pltpu.VMEM((_SC_BC * NSAMPLE, 128), jnp.float32),
            pltpu.VMEM((_SC_BC, 128), jnp.float32),
            pltpu.VMEM((8, 128), jnp.float32),
            pltpu.VMEM((_SC_BC, 256), jnp.float32),
            pltpu.SemaphoreType.DMA,
        ],
    )
    return f(knn_flat, ytab, cent, wxyz)


def _final_body(n_valid, agg_ref, g_ref, b_ref, out_ref):
    m_pad = agg_ref.shape[0]
    rows = jax.lax.broadcasted_iota(jnp.int32, (m_pad, OUT_PLANES), 0) < n_valid
    zero = jnp.float32(0.0)
    hsum = jnp.where(rows, agg_ref[:, 128:192], zero)
    hsq = jnp.where(rows, agg_ref[:, 192:256], zero)
    cnt = jnp.float32(n_valid * NSAMPLE)
    mean = jnp.sum(hsum, axis=0, keepdims=True) / cnt
    var = jnp.sum(hsq, axis=0, keepdims=True) / cnt - mean * mean
    sq = jnp.sqrt(var + 1e-5)
    g = g_ref[...]
    v = jnp.where(g >= 0, agg_ref[:, 0:64], agg_ref[:, 64:128])
    out_ref[...] = jnp.maximum((v - mean) / sq * g + b_ref[...], zero)


def _final_pallas(agg, gamma, beta, n_valid):
    m_pad = agg.shape[0]
    return pl.pallas_call(
        functools.partial(_final_body, n_valid),
        in_specs=[pl.BlockSpec((m_pad, 256), lambda: (0, 0)),
                  pl.BlockSpec((1, OUT_PLANES), lambda: (0, 0)),
                  pl.BlockSpec((1, OUT_PLANES), lambda: (0, 0))],
        out_specs=pl.BlockSpec((m_pad, OUT_PLANES), lambda: (0, 0)),
        out_shape=jax.ShapeDtypeStruct((m_pad, OUT_PLANES), jnp.float32),
    )(agg, gamma.reshape(1, -1), beta.reshape(1, -1))


def kernel(p, x, o, W, gamma, beta):
    idx = _fps_pallas(p, N, M)
    n_p = p[idx]
    knn_full = _knn_pallas(n_p, p)

    n_pad = 50176
    m_pad = knn_full.shape[0]
    xpad = jnp.zeros((n_pad, IN_PLANES), jnp.float32).at[:N].set(x)
    prow = jnp.zeros((n_pad, 16), jnp.float32).at[:N, 0:3].set(p)
    ytab = _ymat_pallas(xpad, W[3:], prow)
    cent = jnp.zeros((m_pad, 128), jnp.float32).at[:M, 0:3].set(n_p)
    wxyz = jnp.zeros((8, 128), jnp.float32).at[0:3, 0:OUT_PLANES].set(W[0:3])
    agg = _scpool_pallas(knn_full.reshape(-1), ytab, cent, wxyz)
    out = _final_pallas(agg, gamma, beta, M)[:M]
    n_o = jnp.array([M], dtype=jnp.int32)
    return (n_p, out, n_o)
